# baseline (speedup 1.0000x reference)
import functools

import jax
import jax.numpy as jnp
from jax import lax
from jax.experimental import pallas as pl
from jax.experimental.pallas import tpu as pltpu
from jax.experimental.pallas import tpu_sc as plsc

NC = 2
NS = 16
NW = NC * NS
LANES = 16
CH = 128

G = 64

def _mesh():
  return plsc.VectorSubcoreMesh(
      core_axis_name="c", subcore_axis_name="s", num_cores=NC, num_subcores=NS)


def _cdiv(a, b):
  return (a + b - 1) // b


def _make_deg_kernel(NR, C1):
  TR = NR // NS

  @functools.partial(
      pl.kernel,
      out_type=jax.ShapeDtypeStruct((NR,), jnp.float32),
      mesh=_mesh(),
      scratch_types=[
          pltpu.VMEM((C1, CH), jnp.int32),
          pltpu.VMEM((CH,), jnp.float32),
          pltpu.VMEM((TR,), jnp.float32),
          pltpu.VMEM_SHARED((NR,), jnp.float32),
      ],
  )
  def deg_kernel(dst_hbm, out_hbm, dst_v, ones_v, init_v, deg_sh):
    cid = lax.axis_index("c")
    sid = lax.axis_index("s")

    @pl.when(cid == 0)
    def _():
      for i in range(CH // LANES):
        ones_v[pl.ds(i * LANES, LANES)] = jnp.ones((LANES,), jnp.float32)
      for i in range(TR // LANES):
        init_v[pl.ds(i * LANES, LANES)] = jnp.ones((LANES,), jnp.float32)
      pltpu.sync_copy(init_v, deg_sh.at[pl.ds(sid * TR, TR)])
      pltpu.sync_copy(dst_hbm.at[sid], dst_v)
      plsc.subcore_barrier()

      def body(j, carry):
        pltpu.sync_copy(ones_v, deg_sh.at[dst_v.at[j]], add=True)
        return carry

      lax.fori_loop(0, C1, body, 0)
      plsc.subcore_barrier()
      pltpu.sync_copy(deg_sh.at[pl.ds(sid * TR, TR)],
                      out_hbm.at[pl.ds(sid * TR, TR)])

  return deg_kernel


def _make_agg_kernel(N, D, NR, C):
  TR = NR // NS
  ZR = 8
  BK = 8
  NB = C // BK
  assert C % BK == 0 and TR % ZR == 0 and NB >= 3

  @functools.partial(
      pl.kernel,
      out_type=jax.ShapeDtypeStruct((NC, NR, D), jnp.float32),
      mesh=_mesh(),
      scratch_types=[
          pltpu.VMEM((2, BK, 2, CH), jnp.int32),
          pltpu.VMEM((2, CH, D), jnp.float32),
          pltpu.VMEM_SHARED((NR, D), jnp.float32),
      ] + [pltpu.SemaphoreType.DMA] * 4,
  )
  def agg_kernel(hs_hbm, idx_hbm, zeros_hbm, out_hbm, idx_v, rows_v, acc_sh,
                 *sems):
    sem_g = sems[:2]
    sem_s = sems[2:]
    cid = lax.axis_index("c")
    sid = lax.axis_index("s")
    wid = sid * NC + cid

    with jax.named_scope("agg_init"):
      pltpu.sync_copy(zeros_hbm.at[pl.ds(sid * TR, TR)],
                      acc_sh.at[pl.ds(sid * TR, TR)])
      plsc.subcore_barrier()

    def gwait(bi, u):
      pltpu.make_async_copy(
          hs_hbm.at[idx_v.at[bi, u % BK, 0]], rows_v.at[u % 2],
          sem_g[u % 2]).wait()

    def gfire(bi, u):
      pltpu.async_copy(
          hs_hbm.at[idx_v.at[bi, u % BK, 0]], rows_v.at[u % 2],
          sem_g[u % 2])

    def sfire(bi, u):
      pltpu.sync_copy(
          rows_v.at[u % 2], acc_sh.at[idx_v.at[bi, u % BK, 1]], add=True)

    _scope_loop = jax.named_scope("agg_loop"); _scope_loop.__enter__()
    pltpu.sync_copy(idx_hbm.at[wid, pl.ds(0, BK)], idx_v.at[0])
    gfire(0, 0)
    pltpu.sync_copy(idx_hbm.at[wid, pl.ds(BK, BK)], idx_v.at[1])
    for u in range(BK):
      if u < BK - 1:
        gfire(0, u + 1)
      else:
        gfire(1, u + 1)
      gwait(0, u)
      sfire(0, u)

    def block(g, carry):
      bi = lax.rem(g, 2)
      bn = lax.rem(g + 1, 2)
      gfire(bi, 1)
      gwait(bi, 0)
      sfire(bi, 0)
      pltpu.sync_copy(
          idx_hbm.at[wid, pl.ds((g + 1) * BK, BK)], idx_v.at[bn])
      for u in range(1, BK):
        if u < BK - 1:
          gfire(bi, u + 1)
        else:
          gfire(bn, u + 1)
        gwait(bi, u)
        sfire(bi, u)
      return carry

    lax.fori_loop(1, NB - 1, block, 0)

    bl = (NB - 1) % 2
    for u in range(BK):
      if u < BK - 1:
        gfire(bl, u + 1)
      gwait(bl, u)
      sfire(bl, u)

    _scope_loop.__exit__(None, None, None)
    with jax.named_scope("agg_out"):
      plsc.subcore_barrier()
      pltpu.sync_copy(acc_sh.at[pl.ds(sid * TR, TR)],
                      out_hbm.at[cid, pl.ds(sid * TR, TR)])

  return agg_kernel


def _tc1_body(x_ref, w1_ref, deg_ref, hs_ref):
  dinv = lax.rsqrt(jnp.maximum(deg_ref[...], 1.0))
  hs_ref[...] = jnp.dot(x_ref[...], w1_ref[...],
                        preferred_element_type=jnp.float32) * dinv


def _tc2_body(N, parts_ref, hs1_ref, deg_ref, b1_ref, w2_ref, hs2_ref):
  dinv = lax.rsqrt(jnp.maximum(deg_ref[...], 1.0))
  agg = parts_ref[0, :N] + parts_ref[1, :N] + hs1_ref[...]
  h = jnp.maximum(agg * dinv + b1_ref[...], 0.0)
  hs2_ref[...] = jnp.dot(h, w2_ref[...],
                         preferred_element_type=jnp.float32) * dinv


def _tc3_body(N, parts_ref, hs2_ref, deg_ref, b2_ref, batch_ref,
              wf1_ref, bf1_ref, wf2_ref, bf2_ref, out_ref):
  dinv = lax.rsqrt(jnp.maximum(deg_ref[...], 1.0))
  h = jnp.maximum(
      (parts_ref[0, :N] + parts_ref[1, :N] + hs2_ref[...]) * dinv
      + b2_ref[...], 0.0)
  gi = lax.broadcasted_iota(jnp.int32, (G, N), 0)
  onehot = (gi == batch_ref[...]).astype(jnp.float32)
  seg = jnp.dot(onehot, h, preferred_element_type=jnp.float32)
  cnt = jnp.sum(onehot, axis=1, keepdims=True)
  p = seg / jnp.maximum(cnt, 1.0)
  o1 = jnp.maximum(
      jnp.dot(p, wf1_ref[...], preferred_element_type=jnp.float32)
      + bf1_ref[...], 0.0)
  out_ref[...] = (jnp.dot(o1, wf2_ref[...], preferred_element_type=jnp.float32)
                  + bf2_ref[...])


def kernel(x, edge_index, batch, W1, b1, W2, b2, Wf1, bf1, Wf2, bf2):
  N, D = x.shape
  H1 = W1.shape[1]
  H2 = Wf1.shape[1]
  E = edge_index.shape[1]

  x = x.astype(jnp.float32)
  src = edge_index[0].astype(jnp.int32)
  dst = edge_index[1].astype(jnp.int32)
  batch2 = batch.astype(jnp.int32).reshape(1, N)

  NRD = _cdiv(N + 1, NS * LANES) * NS * LANES
  NRA = _cdiv(N + 1, NS * 8) * NS * 8

  C = _cdiv(_cdiv(E, NW), CH)
  C = _cdiv(C, 8) * 8
  EP = NW * C * CH
  src_a = jnp.concatenate(
      [src, jnp.zeros((EP - E,), jnp.int32)]).reshape(NW, C, CH)
  dst_a = jnp.concatenate(
      [dst, jnp.full((EP - E,), N, jnp.int32)]).reshape(NW, C, CH)
  idx_a = jnp.stack([src_a, dst_a], axis=2)

  C1 = _cdiv(_cdiv(E, NS), CH)
  EP1 = NS * C1 * CH
  dst_d = jnp.concatenate(
      [dst, jnp.full((EP1 - E,), N, jnp.int32)]).reshape(NS, C1, CH)

  zeros_a = jnp.zeros((NRA, H1), jnp.float32)

  deg_full = _make_deg_kernel(NRD, C1)(dst_d)
  degn = deg_full[:N].reshape(N, 1)

  tc1 = pl.pallas_call(
      _tc1_body,
      out_shape=jax.ShapeDtypeStruct((N, H1), jnp.float32))
  hs1 = tc1(x, W1, degn)

  agg = _make_agg_kernel(N, H1, NRA, C)
  parts1 = agg(hs1, idx_a, zeros_a)

  tc2 = pl.pallas_call(
      functools.partial(_tc2_body, N),
      out_shape=jax.ShapeDtypeStruct((N, H1), jnp.float32))
  hs2 = tc2(parts1, hs1, degn, b1.reshape(1, H1), W2)

  parts2 = agg(hs2, idx_a, zeros_a)

  tc3 = pl.pallas_call(
      functools.partial(_tc3_body, N),
      out_shape=jax.ShapeDtypeStruct((G, 1), jnp.float32))
  out = tc3(parts2, hs2, degn, b2.reshape(1, H1), batch2,
            Wf1, bf1.reshape(1, H2), Wf2, bf2.reshape(1, 1))
  return out


# speedup vs baseline: 1.1384x; 1.1384x over previous
import functools

import jax
import jax.numpy as jnp
from jax import lax
from jax.experimental import pallas as pl
from jax.experimental.pallas import tpu as pltpu
from jax.experimental.pallas import tpu_sc as plsc

NC = 2
NS = 16
NW = NC * NS
LANES = 16
CH = 128

G = 64

def _mesh():
  return plsc.VectorSubcoreMesh(
      core_axis_name="c", subcore_axis_name="s", num_cores=NC, num_subcores=NS)


def _cdiv(a, b):
  return (a + b - 1) // b


def _make_deg_kernel(NR, C1):
  TR = NR // NS

  @functools.partial(
      pl.kernel,
      out_type=jax.ShapeDtypeStruct((NR,), jnp.float32),
      mesh=_mesh(),
      scratch_types=[
          pltpu.VMEM((C1, CH), jnp.int32),
          pltpu.VMEM((CH,), jnp.float32),
          pltpu.VMEM((TR,), jnp.float32),
          pltpu.VMEM_SHARED((NR,), jnp.float32),
      ],
  )
  def deg_kernel(dst_hbm, out_hbm, dst_v, ones_v, init_v, deg_sh):
    cid = lax.axis_index("c")
    sid = lax.axis_index("s")

    @pl.when(cid == 0)
    def _():
      for i in range(CH // LANES):
        ones_v[pl.ds(i * LANES, LANES)] = jnp.ones((LANES,), jnp.float32)
      for i in range(TR // LANES):
        init_v[pl.ds(i * LANES, LANES)] = jnp.ones((LANES,), jnp.float32)
      pltpu.sync_copy(init_v, deg_sh.at[pl.ds(sid * TR, TR)])
      pltpu.sync_copy(dst_hbm.at[sid], dst_v)
      plsc.subcore_barrier()

      def body(j, carry):
        pltpu.sync_copy(ones_v, deg_sh.at[dst_v.at[j]], add=True)
        return carry

      lax.fori_loop(0, C1, body, 0)
      plsc.subcore_barrier()
      pltpu.sync_copy(deg_sh.at[pl.ds(sid * TR, TR)],
                      out_hbm.at[pl.ds(sid * TR, TR)])

  return deg_kernel


def _make_agg_kernel(N, D, NR, C):
  TR = NR // NS
  ZR = 8
  BK = 8
  NB = C // BK
  assert C % BK == 0 and TR % ZR == 0 and NB >= 3

  @functools.partial(
      pl.kernel,
      out_type=jax.ShapeDtypeStruct((NC, NR, D), jnp.float32),
      mesh=_mesh(),
      scratch_types=[
          pltpu.VMEM((2, BK, 2, CH), jnp.int32),
          pltpu.VMEM((2, CH, D), jnp.float32),
          pltpu.VMEM_SHARED((NR, D), jnp.float32),
      ] + [pltpu.SemaphoreType.DMA] * 4,
  )
  def agg_kernel(hs_hbm, idx_hbm, zeros_hbm, out_hbm, idx_v, rows_v, acc_sh,
                 *sems):
    sem_g = sems[:2]
    sem_s = sems[2:]
    cid = lax.axis_index("c")
    sid = lax.axis_index("s")
    wid = sid * NC + cid

    with jax.named_scope("agg_init"):
      pltpu.sync_copy(zeros_hbm.at[pl.ds(sid * TR, TR)],
                      acc_sh.at[pl.ds(sid * TR, TR)])
      plsc.subcore_barrier()

    def gwait(bi, u):
      pltpu.make_async_copy(
          hs_hbm.at[idx_v.at[bi, u % BK, 0]], rows_v.at[u % 2],
          sem_g[u % 2]).wait()

    def gfire(bi, u):
      pltpu.async_copy(
          hs_hbm.at[idx_v.at[bi, u % BK, 0]], rows_v.at[u % 2],
          sem_g[u % 2])

    def sfire(bi, u):
      pltpu.sync_copy(
          rows_v.at[u % 2], acc_sh.at[idx_v.at[bi, u % BK, 1]], add=True)

    _scope_loop = jax.named_scope("agg_loop"); _scope_loop.__enter__()
    pltpu.sync_copy(idx_hbm.at[wid, pl.ds(0, BK)], idx_v.at[0])
    gfire(0, 0)
    pltpu.sync_copy(idx_hbm.at[wid, pl.ds(BK, BK)], idx_v.at[1])
    for u in range(BK):
      if u < BK - 1:
        gfire(0, u + 1)
      else:
        gfire(1, u + 1)
      gwait(0, u)
      sfire(0, u)

    def block(g, carry):
      bi = lax.rem(g, 2)
      bn = lax.rem(g + 1, 2)
      gfire(bi, 1)
      gwait(bi, 0)
      sfire(bi, 0)
      pltpu.sync_copy(
          idx_hbm.at[wid, pl.ds((g + 1) * BK, BK)], idx_v.at[bn])
      for u in range(1, BK):
        if u < BK - 1:
          gfire(bi, u + 1)
        else:
          gfire(bn, u + 1)
        gwait(bi, u)
        sfire(bi, u)
      return carry

    lax.fori_loop(1, NB - 1, block, 0)

    bl = (NB - 1) % 2
    for u in range(BK):
      if u < BK - 1:
        gfire(bl, u + 1)
      gwait(bl, u)
      sfire(bl, u)

    _scope_loop.__exit__(None, None, None)
    with jax.named_scope("agg_out"):
      plsc.subcore_barrier()
      pltpu.sync_copy(acc_sh.at[pl.ds(sid * TR, TR)],
                      out_hbm.at[cid, pl.ds(sid * TR, TR)])

  return agg_kernel


def _tc1_body(N, x_ref, w1_ref, deg_ref, hs_ref):
  dinv = lax.rsqrt(jnp.maximum(deg_ref[...], 1.0))
  hs_ref[pl.ds(0, N), :] = jnp.dot(x_ref[...], w1_ref[...],
                                   preferred_element_type=jnp.float32) * dinv
  hs_ref[pl.ds(N, 8), :] = jnp.zeros((8, x_ref.shape[1]), jnp.float32)


def _tc2_body(N, parts_ref, hs1_ref, deg_ref, b1_ref, w2_ref, hs2_ref):
  dinv = lax.rsqrt(jnp.maximum(deg_ref[...], 1.0))
  agg = parts_ref[0, :N] + parts_ref[1, :N] + hs1_ref[:N]
  h = jnp.maximum(agg * dinv + b1_ref[...], 0.0)
  hs2_ref[pl.ds(0, N), :] = jnp.dot(h, w2_ref[...],
                                    preferred_element_type=jnp.float32) * dinv
  hs2_ref[pl.ds(N, 8), :] = jnp.zeros((8, w2_ref.shape[1]), jnp.float32)


def _tc3_body(N, parts_ref, hs2_ref, deg_ref, b2_ref, batch_ref,
              wf1_ref, bf1_ref, wf2_ref, bf2_ref, out_ref):
  dinv = lax.rsqrt(jnp.maximum(deg_ref[...], 1.0))
  h = jnp.maximum(
      (parts_ref[0, :N] + parts_ref[1, :N] + hs2_ref[:N]) * dinv
      + b2_ref[...], 0.0)
  gi = lax.broadcasted_iota(jnp.int32, (G, N), 0)
  onehot = (gi == batch_ref[...]).astype(jnp.float32)
  seg = jnp.dot(onehot, h, preferred_element_type=jnp.float32)
  cnt = jnp.sum(onehot, axis=1, keepdims=True)
  p = seg / jnp.maximum(cnt, 1.0)
  o1 = jnp.maximum(
      jnp.dot(p, wf1_ref[...], preferred_element_type=jnp.float32)
      + bf1_ref[...], 0.0)
  out_ref[...] = (jnp.dot(o1, wf2_ref[...], preferred_element_type=jnp.float32)
                  + bf2_ref[...])


def kernel(x, edge_index, batch, W1, b1, W2, b2, Wf1, bf1, Wf2, bf2):
  N, D = x.shape
  H1 = W1.shape[1]
  H2 = Wf1.shape[1]
  E = edge_index.shape[1]

  x = x.astype(jnp.float32)
  src = edge_index[0].astype(jnp.int32)
  dst = edge_index[1].astype(jnp.int32)
  batch2 = batch.astype(jnp.int32).reshape(1, N)

  NRD = _cdiv(N + 1, NS * LANES) * NS * LANES
  NRA = _cdiv(N + 1, NS * 8) * NS * 8

  C = _cdiv(_cdiv(E, NW), CH)
  C = _cdiv(C, 8) * 8
  EP = NW * C * CH
  src_a = jnp.concatenate(
      [src, jnp.full((EP - E,), N, jnp.int32)]).reshape(NW, C, CH)
  dst_a = jnp.concatenate(
      [dst, jnp.arange(EP - E, dtype=jnp.int32) % N]).reshape(NW, C, CH)
  idx_a = jnp.stack([src_a, dst_a], axis=2)

  C1 = _cdiv(_cdiv(E, NS), CH)
  EP1 = NS * C1 * CH
  dst_d = jnp.concatenate(
      [dst, N + jnp.arange(EP1 - E, dtype=jnp.int32) % (NRD - N)]
  ).reshape(NS, C1, CH)

  zeros_a = jnp.zeros((NRA, H1), jnp.float32)

  deg_full = _make_deg_kernel(NRD, C1)(dst_d)
  degn = deg_full[:N].reshape(N, 1)

  tc1 = pl.pallas_call(
      functools.partial(_tc1_body, N),
      out_shape=jax.ShapeDtypeStruct((N + 8, H1), jnp.float32))
  hs1 = tc1(x, W1, degn)

  agg = _make_agg_kernel(N, H1, NRA, C)
  parts1 = agg(hs1, idx_a, zeros_a)

  tc2 = pl.pallas_call(
      functools.partial(_tc2_body, N),
      out_shape=jax.ShapeDtypeStruct((N + 8, H1), jnp.float32))
  hs2 = tc2(parts1, hs1, degn, b1.reshape(1, H1), W2)

  parts2 = agg(hs2, idx_a, zeros_a)

  tc3 = pl.pallas_call(
      functools.partial(_tc3_body, N),
      out_shape=jax.ShapeDtypeStruct((G, 1), jnp.float32))
  out = tc3(parts2, hs2, degn, b2.reshape(1, H1), batch2,
            Wf1, bf1.reshape(1, H2), Wf2, bf2.reshape(1, 1))
  return out


# speedup vs baseline: 3.4104x; 2.9958x over previous
import functools

import jax
import jax.numpy as jnp
from jax import lax
from jax.experimental import pallas as pl
from jax.experimental.pallas import tpu as pltpu
from jax.experimental.pallas import tpu_sc as plsc

NC = 2
NS = 16
NW = NC * NS
LANES = 16
CH = 128

G = 64

def _mesh():
  return plsc.VectorSubcoreMesh(
      core_axis_name="c", subcore_axis_name="s", num_cores=NC, num_subcores=NS)


def _cdiv(a, b):
  return (a + b - 1) // b


def _make_deg_kernel(NR, C1):
  TR = NR // NS

  @functools.partial(
      pl.kernel,
      out_type=jax.ShapeDtypeStruct((NR,), jnp.float32),
      mesh=_mesh(),
      scratch_types=[
          pltpu.VMEM((C1, CH), jnp.int32),
          pltpu.VMEM((CH,), jnp.float32),
          pltpu.VMEM((TR,), jnp.float32),
          pltpu.VMEM_SHARED((NR,), jnp.float32),
      ],
  )
  def deg_kernel(dst_hbm, out_hbm, dst_v, ones_v, init_v, deg_sh):
    cid = lax.axis_index("c")
    sid = lax.axis_index("s")

    @pl.when(cid == 0)
    def _():
      for i in range(CH // LANES):
        ones_v[pl.ds(i * LANES, LANES)] = jnp.ones((LANES,), jnp.float32)
      for i in range(TR // LANES):
        init_v[pl.ds(i * LANES, LANES)] = jnp.ones((LANES,), jnp.float32)
      pltpu.sync_copy(init_v, deg_sh.at[pl.ds(sid * TR, TR)])
      pltpu.sync_copy(dst_hbm.at[sid], dst_v)
      plsc.subcore_barrier()

      def body(j, carry):
        pltpu.sync_copy(ones_v, deg_sh.at[dst_v.at[j]], add=True)
        return carry

      lax.fori_loop(0, C1, body, 0)
      plsc.subcore_barrier()
      pltpu.sync_copy(deg_sh.at[pl.ds(sid * TR, TR)],
                      out_hbm.at[pl.ds(sid * TR, TR)])

  return deg_kernel


def _make_agg_kernel(N, D, NR, C):
  TR = NR // NS
  ZR = 8
  BK = 8
  NB = C // BK
  assert C % BK == 0 and TR % ZR == 0 and NB >= 3

  @functools.partial(
      pl.kernel,
      out_type=jax.ShapeDtypeStruct((NC, NR, D), jnp.float32),
      mesh=_mesh(),
      scratch_types=[
          pltpu.VMEM((2, BK, 2, CH), jnp.int32),
          pltpu.VMEM((2, CH, D), jnp.float32),
          pltpu.VMEM_SHARED((NR, D), jnp.float32),
      ] + [pltpu.SemaphoreType.DMA] * 4,
  )
  def agg_kernel(hs_hbm, idx_hbm, zeros_hbm, out_hbm, idx_v, rows_v, acc_sh,
                 *sems):
    sem_g = sems[:2]
    sem_s = sems[2:]
    cid = lax.axis_index("c")
    sid = lax.axis_index("s")
    wid = sid * NC + cid

    with jax.named_scope("agg_init"):
      pltpu.sync_copy(zeros_hbm.at[pl.ds(sid * TR, TR)],
                      acc_sh.at[pl.ds(sid * TR, TR)])
      plsc.subcore_barrier()

    def gwait(bi, u):
      pltpu.make_async_copy(
          hs_hbm.at[idx_v.at[bi, u % BK, 0]], rows_v.at[u % 2],
          sem_g[u % 2]).wait()

    def gfire(bi, u):
      pltpu.async_copy(
          hs_hbm.at[idx_v.at[bi, u % BK, 0]], rows_v.at[u % 2],
          sem_g[u % 2])

    def sfire(bi, u):
      pltpu.sync_copy(
          rows_v.at[u % 2], acc_sh.at[idx_v.at[bi, u % BK, 1]], add=True)

    _scope_loop = jax.named_scope("agg_loop"); _scope_loop.__enter__()
    pltpu.sync_copy(idx_hbm.at[wid, pl.ds(0, BK)], idx_v.at[0])
    gfire(0, 0)
    pltpu.sync_copy(idx_hbm.at[wid, pl.ds(BK, BK)], idx_v.at[1])
    for u in range(BK):
      if u < BK - 1:
        gfire(0, u + 1)
      else:
        gfire(1, u + 1)
      gwait(0, u)
      sfire(0, u)

    def block(g, carry):
      bi = lax.rem(g, 2)
      bn = lax.rem(g + 1, 2)
      gfire(bi, 1)
      gwait(bi, 0)
      sfire(bi, 0)
      pltpu.sync_copy(
          idx_hbm.at[wid, pl.ds((g + 1) * BK, BK)], idx_v.at[bn])
      for u in range(1, BK):
        if u < BK - 1:
          gfire(bi, u + 1)
        else:
          gfire(bn, u + 1)
        gwait(bi, u)
        sfire(bi, u)
      return carry

    lax.fori_loop(1, NB - 1, block, 0)

    bl = (NB - 1) % 2
    for u in range(BK):
      if u < BK - 1:
        gfire(bl, u + 1)
      gwait(bl, u)
      sfire(bl, u)

    _scope_loop.__exit__(None, None, None)
    with jax.named_scope("agg_out"):
      plsc.subcore_barrier()
      pltpu.sync_copy(acc_sh.at[pl.ds(sid * TR, TR)],
                      out_hbm.at[cid, pl.ds(sid * TR, TR)])

  return agg_kernel


def _tc1_body(N, x_ref, w1_ref, deg_ref, hs_ref):
  dinv = lax.rsqrt(jnp.maximum(deg_ref[...], 1.0))
  hs_ref[pl.ds(0, N), :] = jnp.dot(x_ref[...], w1_ref[...],
                                   preferred_element_type=jnp.float32) * dinv
  hs_ref[pl.ds(N, 8), :] = jnp.zeros((8, x_ref.shape[1]), jnp.float32)


def _tc2_body(N, parts_ref, hs1_ref, deg_ref, b1_ref, w2_ref, hs2_ref):
  dinv = lax.rsqrt(jnp.maximum(deg_ref[...], 1.0))
  agg = parts_ref[0, :N] + parts_ref[1, :N] + hs1_ref[:N]
  h = jnp.maximum(agg * dinv + b1_ref[...], 0.0)
  hs2_ref[pl.ds(0, N), :] = jnp.dot(h, w2_ref[...],
                                    preferred_element_type=jnp.float32) * dinv
  hs2_ref[pl.ds(N, 8), :] = jnp.zeros((8, w2_ref.shape[1]), jnp.float32)


def _tc3_body(N, parts_ref, hs2_ref, deg_ref, b2_ref, batch_ref,
              wf1_ref, bf1_ref, wf2_ref, bf2_ref, out_ref):
  dinv = lax.rsqrt(jnp.maximum(deg_ref[...], 1.0))
  h = jnp.maximum(
      (parts_ref[0, :N] + parts_ref[1, :N] + hs2_ref[:N]) * dinv
      + b2_ref[...], 0.0)
  gi = lax.broadcasted_iota(jnp.int32, (G, N), 0)
  onehot = (gi == batch_ref[...]).astype(jnp.float32)
  seg = jnp.dot(onehot, h, preferred_element_type=jnp.float32)
  cnt = jnp.sum(onehot, axis=1, keepdims=True)
  p = seg / jnp.maximum(cnt, 1.0)
  o1 = jnp.maximum(
      jnp.dot(p, wf1_ref[...], preferred_element_type=jnp.float32)
      + bf1_ref[...], 0.0)
  out_ref[...] = (jnp.dot(o1, wf2_ref[...], preferred_element_type=jnp.float32)
                  + bf2_ref[...])


def kernel(x, edge_index, batch, W1, b1, W2, b2, Wf1, bf1, Wf2, bf2):
  N, D = x.shape
  H1 = W1.shape[1]
  H2 = Wf1.shape[1]
  E = edge_index.shape[1]

  x = x.astype(jnp.float32)
  src = edge_index[0].astype(jnp.int32)
  dst = edge_index[1].astype(jnp.int32)
  batch2 = batch.astype(jnp.int32).reshape(1, N)

  NRD = _cdiv(N + 1, NS * LANES) * NS * LANES
  NRA = _cdiv(N + 1, NS * 8) * NS * 8

  C = _cdiv(_cdiv(E, NW), CH)
  C = _cdiv(C, 8) * 8
  EP = NW * C * CH
  src_a = jnp.concatenate(
      [src, jnp.arange(EP - E, dtype=jnp.int32) % N]).reshape(NW, C, CH)
  dst_a = jnp.concatenate(
      [dst, N + jnp.arange(EP - E, dtype=jnp.int32) % (NRA - N)]
  ).reshape(NW, C, CH)
  idx_a = jnp.stack([src_a, dst_a], axis=2)

  C1 = _cdiv(_cdiv(E, NS), CH)
  EP1 = NS * C1 * CH
  dst_d = jnp.concatenate(
      [dst, N + jnp.arange(EP1 - E, dtype=jnp.int32) % (NRD - N)]
  ).reshape(NS, C1, CH)

  zeros_a = jnp.zeros((NRA, H1), jnp.float32)

  deg_full = _make_deg_kernel(NRD, C1)(dst_d)
  degn = deg_full[:N].reshape(N, 1)

  tc1 = pl.pallas_call(
      functools.partial(_tc1_body, N),
      out_shape=jax.ShapeDtypeStruct((N + 8, H1), jnp.float32))
  hs1 = tc1(x, W1, degn)

  agg = _make_agg_kernel(N, H1, NRA, C)
  parts1 = agg(hs1, idx_a, zeros_a)

  tc2 = pl.pallas_call(
      functools.partial(_tc2_body, N),
      out_shape=jax.ShapeDtypeStruct((N + 8, H1), jnp.float32))
  hs2 = tc2(parts1, hs1, degn, b1.reshape(1, H1), W2)

  parts2 = agg(hs2, idx_a, zeros_a)

  tc3 = pl.pallas_call(
      functools.partial(_tc3_body, N),
      out_shape=jax.ShapeDtypeStruct((G, 1), jnp.float32))
  out = tc3(parts2, hs2, degn, b2.reshape(1, H1), batch2,
            Wf1, bf1.reshape(1, H2), Wf2, bf2.reshape(1, 1))
  return out


# speedup vs baseline: 3.5087x; 1.0288x over previous
import functools

import jax
import jax.numpy as jnp
from jax import lax
from jax.experimental import pallas as pl
from jax.experimental.pallas import tpu as pltpu
from jax.experimental.pallas import tpu_sc as plsc

NC = 2
NS = 16
NW = NC * NS
LANES = 16
CH = 128

G = 64

def _mesh():
  return plsc.VectorSubcoreMesh(
      core_axis_name="c", subcore_axis_name="s", num_cores=NC, num_subcores=NS)


def _cdiv(a, b):
  return (a + b - 1) // b


def _make_deg_kernel(NR, C1):
  TR = NR // NS

  @functools.partial(
      pl.kernel,
      out_type=jax.ShapeDtypeStruct((NR,), jnp.float32),
      mesh=_mesh(),
      scratch_types=[
          pltpu.VMEM((C1, CH), jnp.int32),
          pltpu.VMEM((CH,), jnp.float32),
          pltpu.VMEM((TR,), jnp.float32),
          pltpu.VMEM_SHARED((NR,), jnp.float32),
      ],
  )
  def deg_kernel(dst_hbm, out_hbm, dst_v, ones_v, init_v, deg_sh):
    cid = lax.axis_index("c")
    sid = lax.axis_index("s")

    @pl.when(cid == 0)
    def _():
      for i in range(CH // LANES):
        ones_v[pl.ds(i * LANES, LANES)] = jnp.ones((LANES,), jnp.float32)
      for i in range(TR // LANES):
        init_v[pl.ds(i * LANES, LANES)] = jnp.ones((LANES,), jnp.float32)
      pltpu.sync_copy(init_v, deg_sh.at[pl.ds(sid * TR, TR)])
      pltpu.sync_copy(dst_hbm.at[sid], dst_v)
      plsc.subcore_barrier()

      def body(j, carry):
        pltpu.sync_copy(ones_v, deg_sh.at[dst_v.at[j]], add=True)
        return carry

      lax.fori_loop(0, C1, body, 0)
      plsc.subcore_barrier()
      pltpu.sync_copy(deg_sh.at[pl.ds(sid * TR, TR)],
                      out_hbm.at[pl.ds(sid * TR, TR)])

  return deg_kernel


def _make_agg_kernel(N, D, NR, C):
  TR = NR // NS
  ZR = 8
  BK = 8
  NB = C // BK
  assert C % BK == 0 and TR % ZR == 0 and NB >= 3

  @functools.partial(
      pl.kernel,
      out_type=jax.ShapeDtypeStruct((NC, NR, D), jnp.float32),
      mesh=_mesh(),
      scratch_types=[
          pltpu.VMEM((2, BK, 2, CH), jnp.int32),
          pltpu.VMEM((2, CH, D), jnp.float32),
          pltpu.VMEM_SHARED((NR, D), jnp.float32),
      ] + [pltpu.SemaphoreType.DMA] * 4,
  )
  def agg_kernel(hs_hbm, idx_hbm, zeros_hbm, out_hbm, idx_v, rows_v, acc_sh,
                 *sems):
    sem_g = sems[:2]
    sem_s = sems[2:]
    cid = lax.axis_index("c")
    sid = lax.axis_index("s")
    wid = sid * NC + cid

    with jax.named_scope("agg_init"):
      pltpu.sync_copy(zeros_hbm.at[pl.ds(sid * TR, TR)],
                      acc_sh.at[pl.ds(sid * TR, TR)])
      plsc.subcore_barrier()

    def gwait(bi, u):
      pltpu.make_async_copy(
          hs_hbm.at[idx_v.at[bi, u % BK, 0]], rows_v.at[u % 2],
          sem_g[u % 2]).wait()

    def gfire(bi, u):
      pltpu.async_copy(
          hs_hbm.at[idx_v.at[bi, u % BK, 0]], rows_v.at[u % 2],
          sem_g[u % 2])

    def sfire(bi, u):
      pltpu.async_copy(
          rows_v.at[u % 2], acc_sh.at[idx_v.at[bi, u % BK, 1]],
          sem_s[u % 2], add=True)

    def sdrain(bi, u):
      pltpu.make_async_copy(
          rows_v.at[u % 2], acc_sh.at[idx_v.at[bi, 0, 1]],
          sem_s[u % 2]).wait()

    _scope_loop = jax.named_scope("agg_loop"); _scope_loop.__enter__()
    pltpu.sync_copy(idx_hbm.at[wid, pl.ds(0, BK)], idx_v.at[0])
    gfire(0, 0)
    pltpu.sync_copy(idx_hbm.at[wid, pl.ds(BK, BK)], idx_v.at[1])
    for u in range(BK):
      if u >= 1:
        sdrain(0, u - 1)
      if u < BK - 1:
        gfire(0, u + 1)
      else:
        gfire(1, u + 1)
      gwait(0, u)
      sfire(0, u)

    def block(g, carry):
      bi = lax.rem(g, 2)
      bn = lax.rem(g + 1, 2)
      sdrain(bi, BK - 1)
      gfire(bi, 1)
      gwait(bi, 0)
      sfire(bi, 0)
      pltpu.sync_copy(
          idx_hbm.at[wid, pl.ds((g + 1) * BK, BK)], idx_v.at[bn])
      for u in range(1, BK):
        sdrain(bi, u - 1)
        if u < BK - 1:
          gfire(bi, u + 1)
        else:
          gfire(bn, u + 1)
        gwait(bi, u)
        sfire(bi, u)
      return carry

    lax.fori_loop(1, NB - 1, block, 0)

    bl = (NB - 1) % 2
    for u in range(BK):
      sdrain(bl, u - 1)
      if u < BK - 1:
        gfire(bl, u + 1)
      gwait(bl, u)
      sfire(bl, u)
    sdrain(bl, BK - 1)

    _scope_loop.__exit__(None, None, None)
    with jax.named_scope("agg_out"):
      plsc.subcore_barrier()
      pltpu.sync_copy(acc_sh.at[pl.ds(sid * TR, TR)],
                      out_hbm.at[cid, pl.ds(sid * TR, TR)])

  return agg_kernel


def _tc1_body(N, x_ref, w1_ref, deg_ref, hs_ref):
  dinv = lax.rsqrt(jnp.maximum(deg_ref[...], 1.0))
  hs_ref[pl.ds(0, N), :] = jnp.dot(x_ref[...], w1_ref[...],
                                   preferred_element_type=jnp.float32) * dinv
  hs_ref[pl.ds(N, 8), :] = jnp.zeros((8, x_ref.shape[1]), jnp.float32)


def _tc2_body(N, parts_ref, hs1_ref, deg_ref, b1_ref, w2_ref, hs2_ref):
  dinv = lax.rsqrt(jnp.maximum(deg_ref[...], 1.0))
  agg = parts_ref[0, :N] + parts_ref[1, :N] + hs1_ref[:N]
  h = jnp.maximum(agg * dinv + b1_ref[...], 0.0)
  hs2_ref[pl.ds(0, N), :] = jnp.dot(h, w2_ref[...],
                                    preferred_element_type=jnp.float32) * dinv
  hs2_ref[pl.ds(N, 8), :] = jnp.zeros((8, w2_ref.shape[1]), jnp.float32)


def _tc3_body(N, parts_ref, hs2_ref, deg_ref, b2_ref, batch_ref,
              wf1_ref, bf1_ref, wf2_ref, bf2_ref, out_ref):
  dinv = lax.rsqrt(jnp.maximum(deg_ref[...], 1.0))
  h = jnp.maximum(
      (parts_ref[0, :N] + parts_ref[1, :N] + hs2_ref[:N]) * dinv
      + b2_ref[...], 0.0)
  gi = lax.broadcasted_iota(jnp.int32, (G, N), 0)
  onehot = (gi == batch_ref[...]).astype(jnp.float32)
  seg = jnp.dot(onehot, h, preferred_element_type=jnp.float32)
  cnt = jnp.sum(onehot, axis=1, keepdims=True)
  p = seg / jnp.maximum(cnt, 1.0)
  o1 = jnp.maximum(
      jnp.dot(p, wf1_ref[...], preferred_element_type=jnp.float32)
      + bf1_ref[...], 0.0)
  out_ref[...] = (jnp.dot(o1, wf2_ref[...], preferred_element_type=jnp.float32)
                  + bf2_ref[...])


def kernel(x, edge_index, batch, W1, b1, W2, b2, Wf1, bf1, Wf2, bf2):
  N, D = x.shape
  H1 = W1.shape[1]
  H2 = Wf1.shape[1]
  E = edge_index.shape[1]

  x = x.astype(jnp.float32)
  src = edge_index[0].astype(jnp.int32)
  dst = edge_index[1].astype(jnp.int32)
  batch2 = batch.astype(jnp.int32).reshape(1, N)

  NRD = _cdiv(N + 1, NS * LANES) * NS * LANES
  NRA = _cdiv(N + 1, NS * 8) * NS * 8

  C = _cdiv(_cdiv(E, NW), CH)
  C = _cdiv(C, 8) * 8
  EP = NW * C * CH
  src_a = jnp.concatenate(
      [src, jnp.arange(EP - E, dtype=jnp.int32) % N]).reshape(NW, C, CH)
  dst_a = jnp.concatenate(
      [dst, N + jnp.arange(EP - E, dtype=jnp.int32) % (NRA - N)]
  ).reshape(NW, C, CH)
  idx_a = jnp.stack([src_a, dst_a], axis=2)

  C1 = _cdiv(_cdiv(E, NS), CH)
  EP1 = NS * C1 * CH
  dst_d = jnp.concatenate(
      [dst, N + jnp.arange(EP1 - E, dtype=jnp.int32) % (NRD - N)]
  ).reshape(NS, C1, CH)

  zeros_a = jnp.zeros((NRA, H1), jnp.float32)

  deg_full = _make_deg_kernel(NRD, C1)(dst_d)
  degn = deg_full[:N].reshape(N, 1)

  tc1 = pl.pallas_call(
      functools.partial(_tc1_body, N),
      out_shape=jax.ShapeDtypeStruct((N + 8, H1), jnp.float32))
  hs1 = tc1(x, W1, degn)

  agg = _make_agg_kernel(N, H1, NRA, C)
  parts1 = agg(hs1, idx_a, zeros_a)

  tc2 = pl.pallas_call(
      functools.partial(_tc2_body, N),
      out_shape=jax.ShapeDtypeStruct((N + 8, H1), jnp.float32))
  hs2 = tc2(parts1, hs1, degn, b1.reshape(1, H1), W2)

  parts2 = agg(hs2, idx_a, zeros_a)

  tc3 = pl.pallas_call(
      functools.partial(_tc3_body, N),
      out_shape=jax.ShapeDtypeStruct((G, 1), jnp.float32))
  out = tc3(parts2, hs2, degn, b2.reshape(1, H1), batch2,
            Wf1, bf1.reshape(1, H2), Wf2, bf2.reshape(1, 1))
  return out
